# baseline (speedup 1.0000x reference)
import functools

import jax
import jax.numpy as jnp
from jax import lax
from jax.experimental import pallas as pl
from jax.experimental.pallas import tpu as pltpu
from jax.experimental.pallas import tpu_sc as plsc

_COLS = 1024


def _argmax_t_body(a_ref, idx_ref):
    a = a_ref[...]
    m = jnp.max(a, axis=0)
    rows = lax.broadcasted_iota(jnp.int32, a.shape, 0)
    idx_ref[...] = jnp.min(jnp.where(a == m[None, :], rows, 1 << 30), axis=0)


def _make_sc_gather(v, d, b):
    info = plsc.get_sparse_core_info()
    nc, ns = info.num_cores, info.num_subcores
    nw = nc * ns
    b_per_w = b // nw
    chunk = 128
    n_chunks = b_per_w // chunk
    mesh = plsc.VectorSubcoreMesh(core_axis_name="c", subcore_axis_name="s")

    @functools.partial(
        pl.kernel,
        mesh=mesh,
        out_type=jax.ShapeDtypeStruct((b, d), jnp.float32),
        scratch_types=[
            pltpu.VMEM((n_chunks, chunk), jnp.int32),
            pltpu.VMEM((b_per_w, d), jnp.float32),
            pltpu.VMEM_SHARED((v, d), jnp.float32),
            pltpu.SemaphoreType.DMA,
        ],
    )
    def gather_kernel(table_hbm, idx_hbm, out_hbm, idx_v, rows_v, table_sp, sem):
        sid = lax.axis_index("s")
        wid = sid * nc + lax.axis_index("c")
        base = wid * b_per_w
        @pl.when(sid == 0)
        def _():
            pltpu.sync_copy(table_hbm, table_sp)

        for j in range(n_chunks):
            pltpu.sync_copy(idx_hbm.at[pl.ds(base + j * chunk, chunk)], idx_v.at[j])
        plsc.subcore_barrier()
        copies = [
            pltpu.async_copy(
                table_sp.at[idx_v.at[j]],
                rows_v.at[pl.ds(j * chunk, chunk)],
                sem,
            )
            for j in range(n_chunks)
        ]
        for cp in copies:
            cp.wait()
        pltpu.sync_copy(rows_v, out_hbm.at[pl.ds(base, b_per_w)])

    return gather_kernel


def kernel(frag_attr, embedding_weight):
    n, c = frag_attr.shape
    v, d = embedding_weight.shape
    ft = frag_attr.T
    idx = pl.pallas_call(
        _argmax_t_body,
        grid=(n // _COLS,),
        in_specs=[pl.BlockSpec((c, _COLS), lambda i: (0, i))],
        out_specs=pl.BlockSpec((_COLS,), lambda i: (i,)),
        out_shape=jax.ShapeDtypeStruct((n,), jnp.int32),
    )(ft)
    return _make_sc_gather(v, d, n)(embedding_weight, idx)


# speedup vs baseline: 1.0671x; 1.0671x over previous
import functools

import jax
import jax.numpy as jnp
from jax import lax
from jax.experimental import pallas as pl
from jax.experimental.pallas import tpu as pltpu
from jax.experimental.pallas import tpu_sc as plsc

_COLS = 2048


def _argmax_t_body(a_ref, idx_ref):
    a = a_ref[...]
    m = jnp.max(a, axis=0)
    rows = lax.broadcasted_iota(jnp.int32, a.shape, 0)
    idx_ref[...] = jnp.min(jnp.where(a == m[None, :], rows, 1 << 30), axis=0)


def _make_sc_gather(v, d, b):
    info = plsc.get_sparse_core_info()
    nc, ns = info.num_cores, info.num_subcores
    nw = nc * ns
    b_per_w = b // nw
    chunk = 128
    n_chunks = b_per_w // chunk
    mesh = plsc.VectorSubcoreMesh(core_axis_name="c", subcore_axis_name="s")

    @functools.partial(
        pl.kernel,
        mesh=mesh,
        out_type=jax.ShapeDtypeStruct((b, d), jnp.float32),
        scratch_types=[
            pltpu.VMEM((n_chunks, chunk), jnp.int32),
            pltpu.VMEM((b_per_w, d), jnp.float32),
            pltpu.VMEM_SHARED((v, d), jnp.float32),
            pltpu.SemaphoreType.DMA,
        ],
    )
    def gather_kernel(table_hbm, idx_hbm, out_hbm, idx_v, rows_v, table_sp, sem):
        sid = lax.axis_index("s")
        wid = sid * nc + lax.axis_index("c")
        base = wid * b_per_w
        @pl.when(sid == 0)
        def _():
            pltpu.sync_copy(table_hbm, table_sp)

        for j in range(n_chunks):
            pltpu.sync_copy(idx_hbm.at[pl.ds(base + j * chunk, chunk)], idx_v.at[j])
        plsc.subcore_barrier()
        copies = [
            pltpu.async_copy(
                table_sp.at[idx_v.at[j]],
                rows_v.at[pl.ds(j * chunk, chunk)],
                sem,
            )
            for j in range(n_chunks)
        ]
        for cp in copies:
            cp.wait()
        pltpu.sync_copy(rows_v, out_hbm.at[pl.ds(base, b_per_w)])

    return gather_kernel


def kernel(frag_attr, embedding_weight):
    n, c = frag_attr.shape
    v, d = embedding_weight.shape
    ft = frag_attr.T
    idx = pl.pallas_call(
        _argmax_t_body,
        grid=(n // _COLS,),
        in_specs=[pl.BlockSpec((c, _COLS), lambda i: (0, i))],
        out_specs=pl.BlockSpec((_COLS,), lambda i: (i,)),
        out_shape=jax.ShapeDtypeStruct((n,), jnp.int32),
    )(ft)
    return _make_sc_gather(v, d, n)(embedding_weight, idx)


# speedup vs baseline: 1.0891x; 1.0206x over previous
import functools

import jax
import jax.numpy as jnp
from jax import lax
from jax.experimental import pallas as pl
from jax.experimental.pallas import tpu as pltpu
from jax.experimental.pallas import tpu_sc as plsc

_COLS = 2048


def _argmax_t_body(a_ref, idx_ref):
    a = a_ref[...]
    m = jnp.max(a, axis=0)
    rows = lax.broadcasted_iota(jnp.int32, a.shape, 0)
    idx_ref[...] = jnp.min(jnp.where(a == m[None, :], rows, 1 << 30), axis=0)


def _make_sc_gather(v, d, b):
    info = plsc.get_sparse_core_info()
    nc, ns = info.num_cores, info.num_subcores
    nw = nc * ns
    b_per_w = b // nw
    chunk = 128
    n_chunks = b_per_w // chunk
    mesh = plsc.VectorSubcoreMesh(core_axis_name="c", subcore_axis_name="s")

    @functools.partial(
        pl.kernel,
        mesh=mesh,
        out_type=jax.ShapeDtypeStruct((b, d), jnp.float32),
        scratch_types=[
            pltpu.VMEM((n_chunks, chunk), jnp.int32),
            pltpu.VMEM((b_per_w, d), jnp.float32),
            pltpu.VMEM_SHARED((v, d), jnp.float32),
            pltpu.SemaphoreType.DMA,
            pltpu.SemaphoreType.DMA,
            pltpu.SemaphoreType.DMA,
            pltpu.SemaphoreType.DMA,
            pltpu.SemaphoreType.DMA,
        ],
    )
    def gather_kernel(
        table_hbm, idx_hbm, out_hbm, idx_v, rows_v, table_sp, s0, s1, s2, s3, ws
    ):
        gsems = [s0, s1, s2, s3]
        sid = lax.axis_index("s")
        wid = sid * nc + lax.axis_index("c")
        base = wid * b_per_w
        @pl.when(sid == 0)
        def _():
            pltpu.sync_copy(table_hbm, table_sp)

        for j in range(n_chunks):
            pltpu.sync_copy(idx_hbm.at[pl.ds(base + j * chunk, chunk)], idx_v.at[j])
        plsc.subcore_barrier()
        copies = [
            pltpu.async_copy(
                table_sp.at[idx_v.at[j]],
                rows_v.at[pl.ds(j * chunk, chunk)],
                gsems[j],
            )
            for j in range(n_chunks)
        ]
        wcopies = []
        for j, cp in enumerate(copies):
            cp.wait()
            wcopies.append(
                pltpu.async_copy(
                    rows_v.at[pl.ds(j * chunk, chunk)],
                    out_hbm.at[pl.ds(base + j * chunk, chunk)],
                    ws,
                )
            )
        for wc in wcopies:
            wc.wait()

    return gather_kernel


def kernel(frag_attr, embedding_weight):
    n, c = frag_attr.shape
    v, d = embedding_weight.shape
    ft = frag_attr.T
    idx = pl.pallas_call(
        _argmax_t_body,
        grid=(n // _COLS,),
        in_specs=[pl.BlockSpec((c, _COLS), lambda i: (0, i))],
        out_specs=pl.BlockSpec((_COLS,), lambda i: (i,)),
        out_shape=jax.ShapeDtypeStruct((n,), jnp.int32),
    )(ft)
    return _make_sc_gather(v, d, n)(embedding_weight, idx)
